# baseline (speedup 1.0000x reference)
import jax
import jax.numpy as jnp
from jax.experimental import pallas as pl

_N_KEYS = 100000
_BITS = 512
_BLK = 2000
_GRID = _N_KEYS // _BLK


def _tc_body(q_ref, k_ref, o_ref):
    q = q_ref[0, :]
    s = 1.0 - 2.0 * q
    n1 = jnp.sum(q)
    k = k_ref[0]
    dist = n1 + jnp.sum(k * s[None, :], axis=1)
    o_ref[0, 0, :] = jnp.where(dist <= 1.0, 1.0, 0.0).astype(jnp.float32)


def kernel(query, keys):
    q2 = query.reshape(1, _BITS).astype(jnp.float32)
    k3 = keys.reshape(_GRID, _BLK, _BITS)
    out = pl.pallas_call(
        _tc_body,
        grid=(_GRID,),
        in_specs=[
            pl.BlockSpec((1, _BITS), lambda i: (0, 0)),
            pl.BlockSpec((1, _BLK, _BITS), lambda i: (i, 0, 0)),
        ],
        out_specs=pl.BlockSpec((1, 1, _BLK), lambda i: (i, 0, 0)),
        out_shape=jax.ShapeDtypeStruct((_GRID, 1, _BLK), jnp.float32),
    )(q2, k3)
    return out.reshape(_N_KEYS)
